# baseline (speedup 1.0000x reference)
import functools

import jax
import jax.numpy as jnp
from jax import lax
from jax.experimental import pallas as pl
from jax.experimental.pallas import tpu as pltpu
from jax.experimental.pallas import tpu_sc as plsc

B = 4096
L = 200
D = 64
HID = 128
V = 1000000

NW = 32
BPW = B // NW
G = 4
NCH = BPW // G
W = 80
NGATHER = (G * L) // W
LANES = 16
NCOL = D // LANES
RUNROLL = 4

CB = 38400
CB2 = CB // 2
NBLK = -(-V // CB)
VPAD = NBLK * CB


def _linearize_body(t_ref, o_ref):
    t = t_ref[...]
    o_ref[...] = jnp.concatenate(
        [jnp.transpose(t[:, :CB2]), jnp.transpose(t[:, CB2:])], axis=1
    )


def _linearize(tT):
    return pl.pallas_call(
        _linearize_body,
        grid=(NBLK,),
        in_specs=[pl.BlockSpec((D, CB), lambda i: (0, i))],
        out_specs=pl.BlockSpec((CB2, 2 * D), lambda i: (i, 0)),
        out_shape=jax.ShapeDtypeStruct((NBLK * CB2, 2 * D), jnp.float32),
    )(tT)


def _sc_pool_body(idx_hbm, table_hbm, out_hbm,
                  idx_v0, idx_v1, rows_v0, rows_v1, acc_v,
                  sem0, sem1):
    wid = lax.axis_index("s") * 2 + lax.axis_index("c")
    base_row = wid * BPW
    idx_vs = (idx_v0, idx_v1)
    rows_vs = (rows_v0, rows_v1)
    sems = (sem0, sem1)

    def issue(ci, buf):
        row0 = base_row + ci * G
        pltpu.sync_copy(idx_hbm.at[pl.ds(row0 * L, G * L)], idx_vs[buf])
        for k in range(NGATHER):
            pltpu.async_copy(
                table_hbm.at[idx_vs[buf].at[pl.ds(k * W, W)]],
                rows_vs[buf].at[pl.ds(k * W, W)],
                sems[buf],
            )

    def drain(buf):
        pltpu.make_async_copy(
            table_hbm.at[pl.ds(0, G * L)], rows_vs[buf], sems[buf]
        ).wait()

    def process(ci, buf):
        drain(buf)
        rows_v = rows_vs[buf]
        for g in range(G):
            def body(r, carry, g=g):
                out = []
                for c in range(NCOL):
                    a = carry[c]
                    for u in range(RUNROLL):
                        a = a + rows_v[g * L + r * RUNROLL + u,
                                       pl.ds(c * LANES, LANES)]
                    out.append(a)
                return tuple(out)

            zero = jnp.zeros((LANES,), jnp.float32)
            acc = lax.fori_loop(0, L // RUNROLL, body, (zero,) * NCOL)
            for c in range(NCOL):
                acc_v[g, pl.ds(c * LANES, LANES)] = acc[c]
        pltpu.sync_copy(acc_v, out_hbm.at[pl.ds(base_row + ci * G, G)])

    issue(0, 0)

    @pl.loop(0, NCH, step=2)
    def _chunk(ci):
        issue(ci + 1, 1)
        process(ci, 0)

        @pl.when(ci + 2 < NCH)
        def _():
            issue(ci + 2, 0)

        process(ci + 1, 1)


def _sc_pool(idx_perm, table_lin):
    mesh = plsc.VectorSubcoreMesh(core_axis_name="c", subcore_axis_name="s")
    kern = pl.kernel(
        _sc_pool_body,
        out_type=jax.ShapeDtypeStruct((B, D), jnp.float32),
        mesh=mesh,
        scratch_types=[
            pltpu.VMEM((G * L,), jnp.int32),
            pltpu.VMEM((G * L,), jnp.int32),
            pltpu.VMEM((G * L, D), jnp.float32),
            pltpu.VMEM((G * L, D), jnp.float32),
            pltpu.VMEM((G, D), jnp.float32),
            pltpu.SemaphoreType.DMA,
            pltpu.SemaphoreType.DMA,
        ],
        compiler_params=pltpu.CompilerParams(use_tc_tiling_on_sc=False),
    )
    return kern(idx_perm, table_lin)


def _mlp_body(x_ref, w1_ref, b1_ref, w2_ref, b2_ref, o_ref):
    h = lax.dot_general(
        x_ref[...], w1_ref[...], (((1,), (1,)), ((), ())),
        preferred_element_type=jnp.float32,
    )
    h = jnp.tanh(h + b1_ref[...])
    o_ref[...] = jnp.sum(h * w2_ref[...], axis=1, keepdims=True) + b2_ref[...]


def kernel(x, emb_table, W1, b1, W2, b2):
    idx = x.reshape(-1)
    m = lax.rem(idx, jnp.int32(CB))
    idx_perm = (idx - m) + 2 * lax.rem(m, jnp.int32(CB2)) + lax.div(
        m, jnp.int32(CB2))
    table_pairs = _linearize(emb_table.T)
    table_lin = table_pairs.reshape(VPAD, D)
    pooled = _sc_pool(idx_perm, table_lin)
    out = pl.pallas_call(
        _mlp_body,
        out_shape=jax.ShapeDtypeStruct((B, 1), jnp.float32),
    )(pooled, W1, b1.reshape(1, HID), W2, b2.reshape(1, 1))
    return out


# speedup vs baseline: 1.0013x; 1.0013x over previous
import functools

import jax
import jax.numpy as jnp
from jax import lax
from jax.experimental import pallas as pl
from jax.experimental.pallas import tpu as pltpu
from jax.experimental.pallas import tpu_sc as plsc

B = 4096
L = 200
D = 64
HID = 128
V = 1000000

NW = 32
BPW = B // NW
G = 4
NCH = BPW // G
W = 80
NGATHER = (G * L) // W
LANES = 16
NCOL = D // LANES
RUNROLL = 4

CB = 25600
CB2 = CB // 2
NBLK = -(-V // CB)
VPAD = NBLK * CB


def _linearize_body(t_ref, o_ref):
    t = t_ref[...]
    o_ref[...] = jnp.concatenate(
        [jnp.transpose(t[:, :CB2]), jnp.transpose(t[:, CB2:])], axis=1
    )


def _linearize(tT):
    return pl.pallas_call(
        _linearize_body,
        grid=(NBLK,),
        in_specs=[pl.BlockSpec((D, CB), lambda i: (0, i))],
        out_specs=pl.BlockSpec((CB2, 2 * D), lambda i: (i, 0)),
        out_shape=jax.ShapeDtypeStruct((NBLK * CB2, 2 * D), jnp.float32),
    )(tT)


def _sc_pool_body(idx_hbm, table_hbm, out_hbm,
                  idx_v0, idx_v1, rows_v0, rows_v1, acc_v,
                  sem0, sem1):
    wid = lax.axis_index("s") * 2 + lax.axis_index("c")
    base_row = wid * BPW
    idx_vs = (idx_v0, idx_v1)
    rows_vs = (rows_v0, rows_v1)
    sems = (sem0, sem1)

    def issue(ci, buf):
        row0 = base_row + ci * G
        pltpu.sync_copy(idx_hbm.at[pl.ds(row0 * L, G * L)], idx_vs[buf])
        for k in range(NGATHER):
            pltpu.async_copy(
                table_hbm.at[idx_vs[buf].at[pl.ds(k * W, W)]],
                rows_vs[buf].at[pl.ds(k * W, W)],
                sems[buf],
            )

    def drain(buf):
        pltpu.make_async_copy(
            table_hbm.at[pl.ds(0, G * L)], rows_vs[buf], sems[buf]
        ).wait()

    def process(ci, buf):
        drain(buf)
        rows_v = rows_vs[buf]
        for g in range(G):
            def body(r, carry, g=g):
                out = []
                for c in range(NCOL):
                    a = carry[c]
                    for u in range(RUNROLL):
                        a = a + rows_v[g * L + r * RUNROLL + u,
                                       pl.ds(c * LANES, LANES)]
                    out.append(a)
                return tuple(out)

            zero = jnp.zeros((LANES,), jnp.float32)
            acc = lax.fori_loop(0, L // RUNROLL, body, (zero,) * NCOL)
            for c in range(NCOL):
                acc_v[g, pl.ds(c * LANES, LANES)] = acc[c]
        pltpu.sync_copy(acc_v, out_hbm.at[pl.ds(base_row + ci * G, G)])

    issue(0, 0)

    @pl.loop(0, NCH, step=2)
    def _chunk(ci):
        issue(ci + 1, 1)
        process(ci, 0)

        @pl.when(ci + 2 < NCH)
        def _():
            issue(ci + 2, 0)

        process(ci + 1, 1)


def _sc_pool(idx_perm, table_lin):
    mesh = plsc.VectorSubcoreMesh(core_axis_name="c", subcore_axis_name="s")
    kern = pl.kernel(
        _sc_pool_body,
        out_type=jax.ShapeDtypeStruct((B, D), jnp.float32),
        mesh=mesh,
        scratch_types=[
            pltpu.VMEM((G * L,), jnp.int32),
            pltpu.VMEM((G * L,), jnp.int32),
            pltpu.VMEM((G * L, D), jnp.float32),
            pltpu.VMEM((G * L, D), jnp.float32),
            pltpu.VMEM((G, D), jnp.float32),
            pltpu.SemaphoreType.DMA,
            pltpu.SemaphoreType.DMA,
        ],
        compiler_params=pltpu.CompilerParams(use_tc_tiling_on_sc=False),
    )
    return kern(idx_perm, table_lin)


def _mlp_body(x_ref, w1_ref, b1_ref, w2_ref, b2_ref, o_ref):
    h = lax.dot_general(
        x_ref[...], w1_ref[...], (((1,), (1,)), ((), ())),
        preferred_element_type=jnp.float32,
    )
    h = jnp.tanh(h + b1_ref[...])
    o_ref[...] = jnp.sum(h * w2_ref[...], axis=1, keepdims=True) + b2_ref[...]


def kernel(x, emb_table, W1, b1, W2, b2):
    idx = x.reshape(-1)
    m = lax.rem(idx, jnp.int32(CB))
    idx_perm = (idx - m) + 2 * lax.rem(m, jnp.int32(CB2)) + lax.div(
        m, jnp.int32(CB2))
    table_pairs = _linearize(emb_table.T)
    table_lin = table_pairs.reshape(VPAD, D)
    pooled = _sc_pool(idx_perm, table_lin)
    out = pl.pallas_call(
        _mlp_body,
        out_shape=jax.ShapeDtypeStruct((B, 1), jnp.float32),
    )(pooled, W1, b1.reshape(1, HID), W2, b2.reshape(1, 1))
    return out
